# baseline (speedup 1.0000x reference)
import functools

import jax
import jax.numpy as jnp
from jax import lax
from jax.experimental import pallas as pl
from jax.experimental.pallas import tpu as pltpu
from jax.experimental.pallas import tpu_sc as plsc

_NC = 2
_NS = 16
_NW = _NC * _NS

_B = 16384
_EMB = 32
_BPW = _B // _NW

@functools.cache
def _gather_embeddings_kernel():
    mesh = plsc.VectorSubcoreMesh(
        core_axis_name="c", subcore_axis_name="s", num_cores=_NC, num_subcores=_NS
    )

    @functools.partial(
        pl.kernel,
        out_type=(
            jax.ShapeDtypeStruct((_B, _EMB), jnp.float32),
            jax.ShapeDtypeStruct((_B, _EMB), jnp.float32),
        ),
        mesh=mesh,
        compiler_params=pltpu.CompilerParams(use_tc_tiling_on_sc=False),
        scratch_types=[
            pltpu.VMEM((_BPW,), jnp.int32),
            pltpu.VMEM((_BPW, _EMB), jnp.float32),
            pltpu.VMEM((_BPW,), jnp.int32),
            pltpu.VMEM((_BPW, _EMB), jnp.float32),
            pltpu.SemaphoreType.DMA,
            pltpu.SemaphoreType.DMA,
        ],
    )
    def gather_embeddings(
        user_hbm,
        res_hbm,
        utab_hbm,
        rtab_hbm,
        uout_hbm,
        rout_hbm,
        uidx_v,
        urows_v,
        ridx_v,
        rrows_v,
        usem,
        rsem,
    ):
        wid = lax.axis_index("s") * _NC + lax.axis_index("c")
        base = wid * _BPW
        pltpu.sync_copy(user_hbm.at[pl.ds(base, _BPW)], uidx_v)
        pltpu.sync_copy(res_hbm.at[pl.ds(base, _BPW)], ridx_v)
        cu = pltpu.async_copy(utab_hbm.at[uidx_v], urows_v, usem)
        cr = pltpu.async_copy(rtab_hbm.at[ridx_v], rrows_v, rsem)
        cu.wait()
        cr.wait()
        pltpu.sync_copy(urows_v, uout_hbm.at[pl.ds(base, _BPW)])
        pltpu.sync_copy(rrows_v, rout_hbm.at[pl.ds(base, _BPW)])

    return gather_embeddings


_BM = 1024


def _mlp_body(u_ref, r_ref, w1u_ref, w1r_ref, b1_ref, w2_ref, b2_ref, w3_ref, b3_ref, o_ref):
    h = jnp.dot(u_ref[...], w1u_ref[...], preferred_element_type=jnp.float32)
    h = h + jnp.dot(r_ref[...], w1r_ref[...], preferred_element_type=jnp.float32)
    h = jnp.maximum(h + b1_ref[...], 0.0)
    h = jnp.dot(h, w2_ref[...], preferred_element_type=jnp.float32) + b2_ref[...]
    h = jnp.maximum(h, 0.0)
    z = jnp.dot(h, w3_ref[...], preferred_element_type=jnp.float32) + b3_ref[...]
    o_ref[...] = 1.0 / (1.0 + jnp.exp(-z))


def _mlp(u_emb, r_emb, w1u, w1r, b1, w2, b2, w3, b3):
    full = lambda i: (0, 0)
    return pl.pallas_call(
        _mlp_body,
        grid=(_B // _BM,),
        in_specs=[
            pl.BlockSpec((_BM, _EMB), lambda i: (i, 0)),
            pl.BlockSpec((_BM, _EMB), lambda i: (i, 0)),
            pl.BlockSpec((_EMB, 64), full),
            pl.BlockSpec((_EMB, 64), full),
            pl.BlockSpec((1, 64), full),
            pl.BlockSpec((64, 32), full),
            pl.BlockSpec((1, 32), full),
            pl.BlockSpec((32, 1), full),
            pl.BlockSpec((1, 1), full),
        ],
        out_specs=pl.BlockSpec((_BM, 1), lambda i: (i, 0)),
        out_shape=jax.ShapeDtypeStruct((_B, 1), jnp.float32),
    )(u_emb, r_emb, w1u, w1r, b1, w2, b2, w3, b3)


def kernel(user, resource, user_table, res_table, W1, b1, W2, b2, W3, b3):
    u_emb, r_emb = _gather_embeddings_kernel()(user, resource, user_table, res_table)
    return _mlp(
        u_emb,
        r_emb,
        W1[:_EMB],
        W1[_EMB:],
        b1.reshape(1, 64),
        W2,
        b2.reshape(1, 32),
        W3,
        b3.reshape(1, 1),
    )


# speedup vs baseline: 1.6657x; 1.6657x over previous
import functools

import jax
import jax.numpy as jnp
from jax import lax
from jax.experimental import pallas as pl
from jax.experimental.pallas import tpu as pltpu
from jax.experimental.pallas import tpu_sc as plsc

_NC = 2
_NS = 16
_NW = _NC * _NS

_B = 16384
_EMB = 32
_PACK = 128 // _EMB
_NROWS = 1000000
_DT_COLS = 2048
_DT_GRID = 123
_QOFF = _DT_COLS * (_DT_GRID - 1)
_GROWS = _DT_COLS * _DT_GRID
_BPW = _B // _NW


@functools.cache
def _gather_wide_kernel():
    mesh = plsc.VectorSubcoreMesh(
        core_axis_name="c", subcore_axis_name="s", num_cores=_NC, num_subcores=_NS
    )

    @functools.partial(
        pl.kernel,
        out_type=(
            jax.ShapeDtypeStruct((_B, 128), jnp.float32),
            jax.ShapeDtypeStruct((_B, 128), jnp.float32),
        ),
        mesh=mesh,
        compiler_params=pltpu.CompilerParams(use_tc_tiling_on_sc=True),
        scratch_types=[
            pltpu.VMEM((_BPW,), jnp.int32),
            pltpu.VMEM((_BPW, 128), jnp.float32),
            pltpu.SemaphoreType.DMA,
        ],
    )
    def gather_wide(
        urow_hbm,
        rrow_hbm,
        gu_hbm,
        gr_hbm,
        uout_hbm,
        rout_hbm,
        idx_v,
        rows_v,
        sem,
    ):
        wid = lax.axis_index("s") * _NC + lax.axis_index("c")
        base = wid * _BPW
        pltpu.sync_copy(urow_hbm.at[pl.ds(base, _BPW)], idx_v)
        pltpu.async_copy(gu_hbm.at[idx_v], rows_v, sem).wait()
        pltpu.sync_copy(rows_v, uout_hbm.at[pl.ds(base, _BPW)])
        pltpu.sync_copy(rrow_hbm.at[pl.ds(base, _BPW)], idx_v)
        pltpu.async_copy(gr_hbm.at[idx_v], rows_v, sem).wait()
        pltpu.sync_copy(rows_v, rout_hbm.at[pl.ds(base, _BPW)])

    return gather_wide


def _detile_body(u0, u1, u2, u3, r0, r1, r2, r3, yu_ref, yr_ref):
    yu_ref[...] = jnp.concatenate(
        [u0[...].T, u1[...].T, u2[...].T, u3[...].T], axis=1
    )
    yr_ref[...] = jnp.concatenate(
        [r0[...].T, r1[...].T, r2[...].T, r3[...].T], axis=1
    )


def _detile(tab_t_u, tab_t_r):
    in_specs = []
    for p in range(_PACK):
        in_specs.append(
            pl.BlockSpec((_EMB, _DT_COLS), lambda i, p=p: (0, i + (_DT_GRID - 1) * p))
        )
    in_specs = in_specs + in_specs
    return pl.pallas_call(
        _detile_body,
        grid=(_DT_GRID,),
        in_specs=in_specs,
        out_specs=[
            pl.BlockSpec((_DT_COLS, 128), lambda i: (i, 0)),
            pl.BlockSpec((_DT_COLS, 128), lambda i: (i, 0)),
        ],
        out_shape=[jax.ShapeDtypeStruct((_GROWS, 128), jnp.float32)] * 2,
    )(*([tab_t_u] * _PACK + [tab_t_r] * _PACK))


_BM = 1024


def _mlp_body(
    gu_ref, gr_ref, pu_ref, pr_ref, w1u_ref, w1r_ref, b1_ref, w2_ref, b2_ref,
    w3_ref, b3_ref, o_ref
):
    lane_grp = lax.broadcasted_iota(jnp.int32, (_BM, 128), 1) >> 5
    mu = (lane_grp == pu_ref[...]).astype(jnp.float32)
    mr = (lane_grp == pr_ref[...]).astype(jnp.float32)
    h = jnp.dot(gu_ref[...] * mu, w1u_ref[...], preferred_element_type=jnp.float32)
    h = h + jnp.dot(gr_ref[...] * mr, w1r_ref[...], preferred_element_type=jnp.float32)
    h = jnp.maximum(h + b1_ref[...], 0.0)
    h = jnp.dot(h, w2_ref[...], preferred_element_type=jnp.float32) + b2_ref[...]
    h = jnp.maximum(h, 0.0)
    z = jnp.dot(h, w3_ref[...], preferred_element_type=jnp.float32) + b3_ref[...]
    o_ref[...] = 1.0 / (1.0 + jnp.exp(-z))


def _mlp(gu, gr, pu, pr, w1u, w1r, b1, w2, b2, w3, b3):
    full = lambda i: (0, 0)
    return pl.pallas_call(
        _mlp_body,
        grid=(_B // _BM,),
        in_specs=[
            pl.BlockSpec((_BM, 128), lambda i: (i, 0)),
            pl.BlockSpec((_BM, 128), lambda i: (i, 0)),
            pl.BlockSpec((_BM, 1), lambda i: (i, 0)),
            pl.BlockSpec((_BM, 1), lambda i: (i, 0)),
            pl.BlockSpec((128, 64), full),
            pl.BlockSpec((128, 64), full),
            pl.BlockSpec((1, 64), full),
            pl.BlockSpec((64, 32), full),
            pl.BlockSpec((1, 32), full),
            pl.BlockSpec((32, 1), full),
            pl.BlockSpec((1, 1), full),
        ],
        out_specs=pl.BlockSpec((_BM, 1), lambda i: (i, 0)),
        out_shape=jax.ShapeDtypeStruct((_B, 1), jnp.float32),
    )(gu, gr, pu, pr, w1u, w1r, b1, w2, b2, w3, b3)


def kernel(user, resource, user_table, res_table, W1, b1, W2, b2, W3, b3):
    gu_tab, gr_tab = _detile(user_table.T, res_table.T)
    pu_full = jnp.minimum(user // _QOFF, _PACK - 1)
    pr_full = jnp.minimum(resource // _QOFF, _PACK - 1)
    urow = user - pu_full * _QOFF
    rrow = resource - pr_full * _QOFF
    gu, gr = _gather_wide_kernel()(urow, rrow, gu_tab, gr_tab)
    pu = pu_full.reshape(_B, 1)
    pr = pr_full.reshape(_B, 1)
    return _mlp(
        gu,
        gr,
        pu,
        pr,
        jnp.tile(W1[:_EMB], (_PACK, 1)),
        jnp.tile(W1[_EMB:], (_PACK, 1)),
        b1.reshape(1, 64),
        W2,
        b2.reshape(1, 32),
        W3,
        b3.reshape(1, 1),
    )


# speedup vs baseline: 3.5609x; 2.1378x over previous
import functools

import jax
import jax.numpy as jnp
from jax import lax
from jax.experimental import pallas as pl
from jax.experimental.pallas import tpu as pltpu
from jax.experimental.pallas import tpu_sc as plsc

_NC = 2
_NS = 16
_NW = _NC * _NS

_B = 16384
_EMB = 32
_PACK = 128 // _EMB
_NROWS = 1000000
_DT_COLS = 2048
_DT_GRID = 123
_QOFF = _DT_COLS * (_DT_GRID - 1)
_GROWS = _DT_COLS * _DT_GRID
_BPW = _B // _NW


@functools.cache
def _gather_wide_kernel():
    mesh = plsc.VectorSubcoreMesh(
        core_axis_name="c", subcore_axis_name="s", num_cores=_NC, num_subcores=_NS
    )

    @functools.partial(
        pl.kernel,
        out_type=(
            jax.ShapeDtypeStruct((_B, 128), jnp.float32),
            jax.ShapeDtypeStruct((_B, 128), jnp.float32),
        ),
        mesh=mesh,
        compiler_params=pltpu.CompilerParams(use_tc_tiling_on_sc=True),
        scratch_types=[
            pltpu.VMEM((_BPW,), jnp.int32),
            pltpu.VMEM((_BPW, 128), jnp.float32),
            pltpu.SemaphoreType.DMA,
        ],
    )
    def gather_wide(
        urow_hbm,
        rrow_hbm,
        gu_hbm,
        gr_hbm,
        uout_hbm,
        rout_hbm,
        idx_v,
        rows_v,
        sem,
    ):
        wid = lax.axis_index("s") * _NC + lax.axis_index("c")
        base = wid * _BPW
        pltpu.sync_copy(urow_hbm.at[pl.ds(base, _BPW)], idx_v)
        pltpu.async_copy(gu_hbm.at[idx_v], rows_v, sem).wait()
        pltpu.sync_copy(rows_v, uout_hbm.at[pl.ds(base, _BPW)])
        pltpu.sync_copy(rrow_hbm.at[pl.ds(base, _BPW)], idx_v)
        pltpu.async_copy(gr_hbm.at[idx_v], rows_v, sem).wait()
        pltpu.sync_copy(rows_v, rout_hbm.at[pl.ds(base, _BPW)])

    return gather_wide


def _mxu_t(x):
    eye = jnp.eye(128, dtype=jnp.float32)
    return lax.dot_general(
        x, eye, (((0,), (0,)), ((), ())), preferred_element_type=jnp.float32
    )


def _detile_body(u0, u1, u2, u3, r0, r1, r2, r3, yu_ref, yr_ref):
    xu = jnp.concatenate([u0[...], u1[...], u2[...], u3[...]], axis=0)
    xr = jnp.concatenate([r0[...], r1[...], r2[...], r3[...]], axis=0)
    yu_ref[...] = _mxu_t(xu)
    yr_ref[...] = _mxu_t(xr)


def _detile(tab_t_u, tab_t_r):
    in_specs = []
    for p in range(_PACK):
        in_specs.append(
            pl.BlockSpec((_EMB, _DT_COLS), lambda i, p=p: (0, i + (_DT_GRID - 1) * p))
        )
    in_specs = in_specs + in_specs
    return pl.pallas_call(
        _detile_body,
        grid=(_DT_GRID,),
        compiler_params=pltpu.CompilerParams(fuse_transposed_lhs_in_matmul=True),
        in_specs=in_specs,
        out_specs=[
            pl.BlockSpec((_DT_COLS, 128), lambda i: (i, 0)),
            pl.BlockSpec((_DT_COLS, 128), lambda i: (i, 0)),
        ],
        out_shape=[jax.ShapeDtypeStruct((_GROWS, 128), jnp.float32)] * 2,
    )(*([tab_t_u] * _PACK + [tab_t_r] * _PACK))


_BM = 1024


def _mlp_body(
    gu_ref, gr_ref, pu_ref, pr_ref, w1u_ref, w1r_ref, b1_ref, w2_ref, b2_ref,
    w3_ref, b3_ref, o_ref
):
    lane_grp = lax.broadcasted_iota(jnp.int32, (_BM, 128), 1) >> 5
    mu = (lane_grp == pu_ref[...]).astype(jnp.float32)
    mr = (lane_grp == pr_ref[...]).astype(jnp.float32)
    h = jnp.dot(gu_ref[...] * mu, w1u_ref[...], preferred_element_type=jnp.float32)
    h = h + jnp.dot(gr_ref[...] * mr, w1r_ref[...], preferred_element_type=jnp.float32)
    h = jnp.maximum(h + b1_ref[...], 0.0)
    h = jnp.dot(h, w2_ref[...], preferred_element_type=jnp.float32) + b2_ref[...]
    h = jnp.maximum(h, 0.0)
    z = jnp.dot(h, w3_ref[...], preferred_element_type=jnp.float32) + b3_ref[...]
    o_ref[...] = 1.0 / (1.0 + jnp.exp(-z))


def _mlp(gu, gr, pu, pr, w1u, w1r, b1, w2, b2, w3, b3):
    full = lambda i: (0, 0)
    return pl.pallas_call(
        _mlp_body,
        grid=(_B // _BM,),
        in_specs=[
            pl.BlockSpec((_BM, 128), lambda i: (i, 0)),
            pl.BlockSpec((_BM, 128), lambda i: (i, 0)),
            pl.BlockSpec((_BM, 1), lambda i: (i, 0)),
            pl.BlockSpec((_BM, 1), lambda i: (i, 0)),
            pl.BlockSpec((128, 64), full),
            pl.BlockSpec((128, 64), full),
            pl.BlockSpec((1, 64), full),
            pl.BlockSpec((64, 32), full),
            pl.BlockSpec((1, 32), full),
            pl.BlockSpec((32, 1), full),
            pl.BlockSpec((1, 1), full),
        ],
        out_specs=pl.BlockSpec((_BM, 1), lambda i: (i, 0)),
        out_shape=jax.ShapeDtypeStruct((_B, 1), jnp.float32),
    )(gu, gr, pu, pr, w1u, w1r, b1, w2, b2, w3, b3)


def kernel(user, resource, user_table, res_table, W1, b1, W2, b2, W3, b3):
    gu_tab, gr_tab = _detile(user_table.T, res_table.T)
    pu_full = jnp.minimum(user // _QOFF, _PACK - 1)
    pr_full = jnp.minimum(resource // _QOFF, _PACK - 1)
    urow = user - pu_full * _QOFF
    rrow = resource - pr_full * _QOFF
    gu, gr = _gather_wide_kernel()(urow, rrow, gu_tab, gr_tab)
    pu = pu_full.reshape(_B, 1)
    pr = pr_full.reshape(_B, 1)
    return _mlp(
        gu,
        gr,
        pu,
        pr,
        jnp.tile(W1[:_EMB], (_PACK, 1)),
        jnp.tile(W1[_EMB:], (_PACK, 1)),
        b1.reshape(1, 64),
        W2,
        b2.reshape(1, 32),
        W3,
        b3.reshape(1, 1),
    )
